# baseline (device time: 13018 ns/iter reference)
import jax
import jax.numpy as jnp
from jax import lax
from jax.experimental import pallas as pl
from jax.experimental.pallas import tpu as pltpu

Z = 4


def kernel(x):
    m, n = x.shape
    b = n // Z
    h = m // 2

    def body(x_ref, out_ref, stage_ref, zsend, xsend, zrecv, xrecv):
        my_x = lax.axis_index("x")
        my_y = lax.axis_index("y")
        my_z = lax.axis_index("z")
        row0 = my_x * h
        prow0 = (1 - my_x) * h

        stage_ref[...] = x_ref[...].astype(jnp.bfloat16)

        barrier_sem = pltpu.get_barrier_semaphore()
        for d in range(1, Z):
            peer = lax.rem(my_z + d, Z)
            for px in (my_x, 1 - my_x):
                pl.semaphore_signal(
                    barrier_sem, inc=1,
                    device_id=(px, my_y, peer),
                    device_id_type=pl.DeviceIdType.MESH,
                )
        pl.semaphore_wait(barrier_sem, 2 * (Z - 1))

        sends = []
        for d in range(1, Z):
            peer = lax.rem(my_z + d, Z)
            for send_sems, recv_sems, px in (
                (zsend, zrecv, my_x),
                (xsend, xrecv, 1 - my_x),
            ):
                rdma = pltpu.make_async_remote_copy(
                    src_ref=stage_ref.at[pl.ds(row0, h), pl.ds(peer * b, b)],
                    dst_ref=out_ref.at[pl.ds(my_z * m + row0, h), :],
                    send_sem=send_sems.at[d - 1],
                    recv_sem=recv_sems.at[d - 1],
                    device_id=(px, my_y, peer),
                    device_id_type=pl.DeviceIdType.MESH,
                )
                rdma.start()
                sends.append(rdma)

        out_ref[pl.ds(my_z * m, m), :] = stage_ref[:, pl.ds(my_z * b, b)]

        for d in range(1, Z):
            src_z = lax.rem(my_z - d + Z, Z)
            for recv_sems, r0 in ((zrecv, row0), (xrecv, prow0)):
                recv = pltpu.make_async_remote_copy(
                    src_ref=out_ref.at[pl.ds(src_z * m + r0, h), :],
                    dst_ref=out_ref.at[pl.ds(src_z * m + r0, h), :],
                    send_sem=zsend.at[d - 1],
                    recv_sem=recv_sems.at[d - 1],
                    device_id=(my_x, my_y, src_z),
                    device_id_type=pl.DeviceIdType.MESH,
                )
                recv.wait_recv()

        for rdma in sends:
            rdma.wait_send()

    out_shape = jax.ShapeDtypeStruct((Z * m, b), jnp.bfloat16)
    return pl.pallas_call(
        body,
        out_shape=out_shape,
        in_specs=[pl.BlockSpec(memory_space=pltpu.VMEM)],
        out_specs=pl.BlockSpec(memory_space=pltpu.VMEM),
        scratch_shapes=[
            pltpu.VMEM((m, n), jnp.bfloat16),
            pltpu.SemaphoreType.DMA((Z - 1,)),
            pltpu.SemaphoreType.DMA((Z - 1,)),
            pltpu.SemaphoreType.DMA((Z - 1,)),
            pltpu.SemaphoreType.DMA((Z - 1,)),
        ],
        compiler_params=pltpu.CompilerParams(collective_id=0),
    )(x)


# device time: 11842 ns/iter; 1.0993x vs baseline; 1.0993x over previous
import jax
import jax.numpy as jnp
from jax import lax
from jax.experimental import pallas as pl
from jax.experimental.pallas import tpu as pltpu

Z = 4


def kernel(x):
    m, n = x.shape
    b = n // Z

    def body(x_ref, out_ref, stage_ref, send_sems, recv_sems):
        my_x = lax.axis_index("x")
        my_y = lax.axis_index("y")
        my_z = lax.axis_index("z")

        stage_ref[...] = x_ref[...].astype(jnp.bfloat16)

        barrier_sem = pltpu.get_barrier_semaphore()
        for d in range(1, Z):
            pl.semaphore_signal(
                barrier_sem, inc=1,
                device_id=(my_x, my_y, lax.rem(my_z + d, Z)),
                device_id_type=pl.DeviceIdType.MESH,
            )
        pl.semaphore_wait(barrier_sem, Z - 1)

        for z in range(Z):

            @pl.when(my_z == z)
            def _(z=z):
                dests = sorted((p for p in range(Z) if p != z),
                               key=lambda p: -abs(p - z))
                sends = []
                for p in dests:
                    d = (p - z) % Z
                    rdma = pltpu.make_async_remote_copy(
                        src_ref=stage_ref.at[:, p * b:(p + 1) * b],
                        dst_ref=out_ref.at[z * m:(z + 1) * m, :],
                        send_sem=send_sems.at[d - 1],
                        recv_sem=recv_sems.at[d - 1],
                        device_id=(my_x, my_y, p),
                        device_id_type=pl.DeviceIdType.MESH,
                    )
                    rdma.start()
                    sends.append(rdma)

                out_ref[z * m:(z + 1) * m, :] = stage_ref[:, z * b:(z + 1) * b]

                for d in range(1, Z):
                    s = (z - d) % Z
                    recv = pltpu.make_async_remote_copy(
                        src_ref=out_ref.at[s * m:(s + 1) * m, :],
                        dst_ref=out_ref.at[s * m:(s + 1) * m, :],
                        send_sem=send_sems.at[d - 1],
                        recv_sem=recv_sems.at[d - 1],
                        device_id=(my_x, my_y, s),
                        device_id_type=pl.DeviceIdType.MESH,
                    )
                    recv.wait_recv()

                for rdma in sends:
                    rdma.wait_send()

    out_shape = jax.ShapeDtypeStruct((Z * m, b), jnp.bfloat16)
    return pl.pallas_call(
        body,
        out_shape=out_shape,
        in_specs=[pl.BlockSpec(memory_space=pltpu.VMEM)],
        out_specs=pl.BlockSpec(memory_space=pltpu.VMEM),
        scratch_shapes=[
            pltpu.VMEM((m, n), jnp.bfloat16),
            pltpu.SemaphoreType.DMA((Z - 1,)),
            pltpu.SemaphoreType.DMA((Z - 1,)),
        ],
        compiler_params=pltpu.CompilerParams(collective_id=0),
    )(x)
